# baseline (speedup 1.0000x reference)
import functools

import jax
import jax.numpy as jnp
from jax import lax
from jax.experimental import pallas as pl
from jax.experimental.pallas import tpu as pltpu

B = 1024
E = 512
KQ = 64
N = 1024
QS = KQ * N
KNN = 100
TEMP = 0.07
NCLS = 1000
G = 16
NG = QS // G
NGB = N // G


def _mm_body(ptr_ref, feat_ref, lab_ref, qf_ref, ql_ref,
             sim_ref, gm_ref, nqf_ref, nql_ref, nptr_ref):
    k = pl.program_id(0)
    f = feat_ref[...]
    q = qf_ref[0]
    s = lax.dot_general(f, q, (((1,), (1,)), ((), ())),
                        preferred_element_type=jnp.float32)
    sim_ref[...] = s
    gm_ref[0] = jnp.max(s.reshape(B, NGB, G), axis=-1)
    ptr = ptr_ref[0]
    sel = k == ptr
    nqf_ref[0] = jnp.where(sel, f, q)
    nql_ref[0] = jnp.where(sel, lab_ref[...], ql_ref[0])

    @pl.when(k == 0)
    def _():
        nptr_ref[0] = (ptr + 1) % KQ


def _matmul_stage(features, labels, queue_features, queue_labels, queue_ptr):
    lab2 = labels.reshape(8, 128)
    ql3 = queue_labels.reshape(KQ, 8, 128)
    out_shapes = (
        jax.ShapeDtypeStruct((B, QS), jnp.float32),
        jax.ShapeDtypeStruct((KQ, B, NGB), jnp.float32),
        jax.ShapeDtypeStruct((KQ, N, E), jnp.float32),
        jax.ShapeDtypeStruct((KQ, 8, 128), jnp.int32),
        jax.ShapeDtypeStruct((1,), jnp.int32),
    )
    grid = (KQ,)
    sim, gm, nqf, nql3, nptr = pl.pallas_call(
        _mm_body,
        grid=grid,
        in_specs=[
            pl.BlockSpec(memory_space=pltpu.SMEM),
            pl.BlockSpec((B, E), lambda k: (0, 0)),
            pl.BlockSpec((8, 128), lambda k: (0, 0)),
            pl.BlockSpec((1, N, E), lambda k: (k, 0, 0)),
            pl.BlockSpec((1, 8, 128), lambda k: (k, 0, 0)),
        ],
        out_specs=(
            pl.BlockSpec((B, N), lambda k: (0, k)),
            pl.BlockSpec((1, B, NGB), lambda k: (k, 0, 0)),
            pl.BlockSpec((1, N, E), lambda k: (k, 0, 0)),
            pl.BlockSpec((1, 8, 128), lambda k: (k, 0, 0)),
            pl.BlockSpec(memory_space=pltpu.SMEM),
        ),
        out_shape=out_shapes,
    )(queue_ptr, features, lab2, queue_features, ql3)
    gm = gm.transpose(1, 0, 2).reshape(B, NG)
    return sim, gm, nqf, nql3.reshape(KQ, N), nptr


def kernel(features, labels, queue_features, queue_labels, queue_ptr):
    sim, gm, nqf, nql, nptr = _matmul_stage(
        features, labels, queue_features, queue_labels, queue_ptr)
    sim_weight, sim_indices = lax.top_k(sim, k=KNN)
    sim_weight = jnp.exp(sim_weight / TEMP)
    qlf = queue_labels.reshape(QS)
    sim_labels = jnp.take(qlf, sim_indices, axis=0)
    one_hot = jax.nn.one_hot(sim_labels, NCLS, dtype=sim_weight.dtype, axis=-1)
    pred_scores = jnp.sum(one_hot * sim_weight[..., None], axis=1)
    pred_labels = jnp.argmax(pred_scores, axis=-1)
    accuracy = jnp.mean((pred_labels == labels).astype(jnp.float32))
    return accuracy, nqf, nql, nptr
